# Initial kernel scaffold; baseline (speedup 1.0000x reference)
import jax
import jax.numpy as jnp
from jax.experimental import pallas as pl


def kernel(x, edge_index, W1a, b1a, W2a, b2a, W1b, b1b, W2b, b2b, Wfc, bfc):
    raise NotImplementedError("write your pallas kernel here")


# speedup vs baseline: 6.6377x; 6.6377x over previous
import functools

import jax
import jax.numpy as jnp
from jax import lax
from jax.experimental import pallas as pl
from jax.experimental.pallas import tpu as pltpu
from jax.experimental.pallas import tpu_sc as plsc

N = 10000
D = 128
E = 320000
NC = 2
NS = 16
NW = NC * NS
EPT = E // NW
CH = 80
NCH = EPT // CH
NP = 10240
RPT = NP // NS


def _sc_scatter_add(h, src3, dst3, zrows):
    mesh = plsc.VectorSubcoreMesh(
        core_axis_name="c", subcore_axis_name="s", num_cores=NC, num_subcores=NS
    )

    @functools.partial(
        pl.kernel,
        out_type=jax.ShapeDtypeStruct((NC, NP, D), jnp.float32),
        mesh=mesh,
        scratch_types=[
            pltpu.VMEM((NCH, CH), jnp.int32),
            pltpu.VMEM((NCH, CH), jnp.int32),
            pltpu.VMEM((CH, D), jnp.float32),
            pltpu.VMEM_SHARED((NP, D), jnp.float32),
            pltpu.SemaphoreType.DMA,
        ],
    )
    def k(h_hbm, src_hbm, dst_hbm, z_hbm, out_hbm,
          src_v, dst_v, rows_v, acc_sh, sem):
        c = lax.axis_index("c")
        s = lax.axis_index("s")
        wid = c * NS + s
        pltpu.sync_copy(src_hbm.at[wid], src_v)
        pltpu.sync_copy(dst_hbm.at[wid], dst_v)
        pltpu.sync_copy(z_hbm, acc_sh.at[pl.ds(s * RPT, RPT)])
        plsc.subcore_barrier()

        def body(j, carry):
            pltpu.async_copy(h_hbm.at[src_v.at[j]], rows_v, sem).wait()
            pltpu.sync_copy(rows_v, acc_sh.at[dst_v.at[j]], add=True)
            return carry

        lax.fori_loop(0, NCH, body, 0)
        plsc.subcore_barrier()
        pltpu.sync_copy(acc_sh.at[pl.ds(s * RPT, RPT)],
                        out_hbm.at[c, pl.ds(s * RPT, RPT)])

    return k(h, src3, dst3, zrows)


def _mlp(h, p, W1, b1, W2, b2):
    def body(h_ref, p_ref, w1_ref, b1_ref, w2_ref, b2_ref, o_ref):
        u = h_ref[...] + p_ref[0, :N] + p_ref[1, :N]
        t = jnp.dot(u, w1_ref[...], preferred_element_type=jnp.float32)
        t = jnp.maximum(t + b1_ref[...], 0.0)
        v = jnp.dot(t, w2_ref[...], preferred_element_type=jnp.float32)
        o_ref[...] = jnp.maximum(v + b2_ref[...], 0.0)

    return pl.pallas_call(
        body,
        out_shape=jax.ShapeDtypeStruct((N, D), jnp.float32),
    )(h, p, W1, b1, W2, b2)


def _mlp_fc(h, p, W1, b1, W2, b2, Wfcp, bfcp):
    def body(h_ref, p_ref, w1_ref, b1_ref, w2_ref, b2_ref,
             wfc_ref, bfc_ref, o_ref):
        u = h_ref[...] + p_ref[0, :N] + p_ref[1, :N]
        t = jnp.dot(u, w1_ref[...], preferred_element_type=jnp.float32)
        t = jnp.maximum(t + b1_ref[...], 0.0)
        v = jnp.dot(t, w2_ref[...], preferred_element_type=jnp.float32)
        h2 = jnp.maximum(v + b2_ref[...], 0.0)
        o = jnp.dot(h2, wfc_ref[...], preferred_element_type=jnp.float32)
        o_ref[...] = o + bfc_ref[...]

    return pl.pallas_call(
        body,
        out_shape=jax.ShapeDtypeStruct((N, D), jnp.float32),
    )(h, p, W1, b1, W2, b2, Wfcp, bfcp)


def kernel(x, edge_index, W1a, b1a, W2a, b2a, W1b, b1b, W2b, b2b, Wfc, bfc):
    src3 = edge_index[0].astype(jnp.int32).reshape(NW, NCH, CH)
    dst3 = edge_index[1].astype(jnp.int32).reshape(NW, NCH, CH)
    zrows = jnp.zeros((RPT, D), jnp.float32)
    b1a2, b2a2, b1b2, b2b2 = (b.reshape(1, D) for b in (b1a, b2a, b1b, b2b))
    Wfcp = jnp.pad(Wfc, ((0, 0), (0, D - Wfc.shape[1])))
    bfcp = jnp.pad(bfc, (0, D - bfc.shape[0])).reshape(1, D)

    p1 = _sc_scatter_add(x, src3, dst3, zrows)
    h1 = _mlp(x, p1, W1a, b1a2, W2a, b2a2)
    p2 = _sc_scatter_add(h1, src3, dst3, zrows)
    o = _mlp_fc(h1, p2, W1b, b1b2, W2b, b2b2, Wfcp, bfcp)
    return o[:, :3]
